# baseline (speedup 1.0000x reference)
import jax
import jax.numpy as jnp
from jax import lax
from jax.experimental import pallas as pl
from jax.experimental.pallas import tpu as pltpu
from jax.experimental.pallas import tpu_sc as plsc

MAXLEN = 50
EMBED_DIM = 64
BATCH = 16384

NC = 2
NS = 16
NW = NC * NS
RPW = BATCH // NW
HALF = 256
NBT = RPW // 128
NV = EMBED_DIM // 16


def _sc_body(x_hbm, c_hbm, tok_hbm, cat_hbm, pos_hbm, out_hbm,
             xidx, cidx, tokbuf, obuf, catbuf, posbuf, gsem, osem, isem):
    wid = lax.axis_index("s") * NC + lax.axis_index("c")
    bbase = wid * RPW

    pltpu.sync_copy(pos_hbm, posbuf)
    pltpu.sync_copy(c_hbm.at[pl.ds(wid * NBT, NBT)], cidx)
    cat_cps = [pltpu.async_copy(cat_hbm.at[cidx.at[j]],
                                catbuf.at[pl.ds(j * 128, 128)], gsem[0])
               for j in range(NBT)]
    for cp in cat_cps:
        cp.wait()

    lane = lax.iota(jnp.int32, 16)
    dt_v = [2 * v + (lane >> 3) for v in range(NV)]
    di_v = lane & 7
    btl_c = [jnp.full((16,), btl, jnp.int32) for btl in range(2)]

    def fire_gathers(half):
        for j in range(2):
            pltpu.async_copy(tok_hbm.at[xidx[half].at[j]],
                             tokbuf[half].at[pl.ds(j * 128, 128)],
                             gsem[half])

    def body(l, half, outwait=True, prefetch=True):
        for j in range(2):
            pltpu.make_async_copy(tok_hbm.at[xidx[half].at[j]],
                                  tokbuf[half].at[pl.ds(j * 128, 128)],
                                  gsem[half]).wait()
        if prefetch:
            pltpu.async_copy(x_hbm.at[l + 1, pl.ds(wid * NBT + half * 2, 2)],
                             xidx[half], isem[half])
        if outwait:
            for dt in range(8):
                pltpu.make_async_copy(
                    obuf[half].at[dt, pl.ds(0, 2), :, pl.ds(0, 128)],
                    out_hbm.at[l, dt, pl.ds(wid * NBT + half * 2, 2)],
                    osem[half]).wait()

        posv = [posbuf[l, pl.ds(v * 16, 16)] for v in range(NV)]

        @plsc.parallel_loop(0, 128, 1, unroll=6)
        def rbody(r):
            bi_s = jnp.full((16,), r, jnp.int32)
            for btl in range(2):
                row = btl * 128 + r
                rg = half * HALF + row
                for v in range(NV):
                    sl = pl.ds(v * 16, 16)
                    val = tokbuf[half][row, sl] + posv[v] + catbuf[rg, sl]
                    plsc.store_scatter(obuf[half],
                                       [dt_v[v], btl_c[btl], di_v, bi_s], val)

        for dt in range(8):
            pltpu.async_copy(
                obuf[half].at[dt, pl.ds(0, 2), :, pl.ds(0, 128)],
                out_hbm.at[l, dt, pl.ds(wid * NBT + half * 2, 2)],
                osem[half])
        if prefetch:
            pltpu.make_async_copy(
                x_hbm.at[l + 1, pl.ds(wid * NBT + half * 2, 2)],
                xidx[half], isem[half]).wait()
            fire_gathers(half)

    for half in range(2):
        pltpu.sync_copy(x_hbm.at[0, pl.ds(wid * NBT + half * 2, 2)],
                        xidx[half])
        fire_gathers(half)
    body(0, 0, outwait=False)
    body(0, 1, outwait=False)

    def lstep(l, carry):
        body(l, 0)
        body(l, 1)
        return carry

    lax.fori_loop(1, MAXLEN - 1, lstep, 0)
    body(MAXLEN - 1, 0, prefetch=False)
    body(MAXLEN - 1, 1, prefetch=False)
    for half in range(2):
        for dt in range(8):
            pltpu.make_async_copy(
                obuf[half].at[dt, pl.ds(0, 2), :, pl.ds(0, 128)],
                out_hbm.at[MAXLEN - 1, dt,
                           pl.ds(wid * NBT + half * 2, 2)],
                osem[half]).wait()


def kernel(X, c, token_table, category_table, pos_table):
    xT = X.T.reshape(MAXLEN, BATCH // 128, 128)
    c2 = c.reshape(BATCH // 128, 128)
    mesh = plsc.VectorSubcoreMesh(core_axis_name="c", subcore_axis_name="s")
    run = pl.kernel(
        _sc_body,
        mesh=mesh,
        compiler_params=pltpu.CompilerParams(use_tc_tiling_on_sc=False,
                                            needs_layout_passes=False),
        out_type=jax.ShapeDtypeStruct((MAXLEN, 8, 128, 8, 128), jnp.float32),
        scratch_types=[
            [pltpu.VMEM((2, 128), jnp.int32) for _ in range(2)],
            pltpu.VMEM((NBT, 128), jnp.int32),
            [pltpu.VMEM((HALF, EMBED_DIM), jnp.float32) for _ in range(2)],
            [pltpu.VMEM((8, 3, 8, 129), jnp.float32) for _ in range(2)],
            pltpu.VMEM((RPW, EMBED_DIM), jnp.float32),
            pltpu.VMEM((MAXLEN, EMBED_DIM), jnp.float32),
            [pltpu.SemaphoreType.DMA for _ in range(2)],
            [pltpu.SemaphoreType.DMA for _ in range(2)],
            [pltpu.SemaphoreType.DMA for _ in range(2)],
        ],
    )
    out5 = run(xT, c2, token_table, category_table, pos_table)
    return out5.transpose(2, 4, 0, 1, 3).reshape(BATCH, MAXLEN, EMBED_DIM)


# speedup vs baseline: 1.0360x; 1.0360x over previous
import jax
import jax.numpy as jnp
from jax import lax
from jax.experimental import pallas as pl
from jax.experimental.pallas import tpu as pltpu
from jax.experimental.pallas import tpu_sc as plsc

MAXLEN = 50
EMBED_DIM = 64
BATCH = 16384

NC = 2
NS = 16
NW = NC * NS
RPW = BATCH // NW
HALF = 256
NBT = RPW // 128
NV = EMBED_DIM // 16


def _sc_body(x_hbm, c_hbm, tok_hbm, cat_hbm, pos_hbm, out_hbm,
             xidx, cidx, tokbuf, obuf, catbuf, posbuf, gsem, osem, isem):
    wid = lax.axis_index("s") * NC + lax.axis_index("c")

    pltpu.sync_copy(pos_hbm, posbuf)
    pltpu.sync_copy(c_hbm.at[pl.ds(wid * NBT, NBT)], cidx)
    cat_cps = [pltpu.async_copy(cat_hbm.at[cidx.at[j]],
                                catbuf.at[pl.ds(j * 128, 128)], gsem[0])
               for j in range(NBT)]
    for cp in cat_cps:
        cp.wait()

    lane = lax.iota(jnp.int32, 16)
    dt_v = [2 * v + (lane >> 3) for v in range(NV)]
    di_v = lane & 7
    btl_c = [jnp.full((16,), btl, jnp.int32) for btl in range(2)]

    def fire_gathers(half):
        for j in range(2):
            pltpu.async_copy(tok_hbm.at[xidx[half].at[j]],
                             tokbuf[half].at[pl.ds(j * 128, 128)],
                             gsem[half])

    def body(l, half, outwait=True, prefetch=True):
        for j in range(2):
            pltpu.make_async_copy(tok_hbm.at[xidx[half].at[j]],
                                  tokbuf[half].at[pl.ds(j * 128, 128)],
                                  gsem[half]).wait()
        if prefetch:
            pltpu.async_copy(x_hbm.at[l + 1, pl.ds(wid * NBT + half * 2, 2)],
                             xidx[half], isem[half])
        if outwait:
            for dt in range(8):
                pltpu.make_async_copy(
                    obuf[half].at[dt, pl.ds(0, 2), :, pl.ds(0, 128)],
                    out_hbm.at[l, dt, pl.ds(wid * NBT + half * 2, 2)],
                    osem[half]).wait()

        posv = [posbuf[l, pl.ds(v * 16, 16)] for v in range(NV)]

        @plsc.parallel_loop(0, 128, 1, unroll=4)
        def rbody(r):
            bi_s = jnp.full((16,), r, jnp.int32)
            for btl in range(2):
                row = btl * 128 + r
                rg = half * HALF + row
                for v in range(NV):
                    sl = pl.ds(v * 16, 16)
                    val = tokbuf[half][row, sl] + posv[v] + catbuf[rg, sl]
                    plsc.store_scatter(obuf[half],
                                       [dt_v[v], btl_c[btl], di_v, bi_s], val)

        for dt in range(8):
            pltpu.async_copy(
                obuf[half].at[dt, pl.ds(0, 2), :, pl.ds(0, 128)],
                out_hbm.at[l, dt, pl.ds(wid * NBT + half * 2, 2)],
                osem[half])
        if prefetch:
            pltpu.make_async_copy(
                x_hbm.at[l + 1, pl.ds(wid * NBT + half * 2, 2)],
                xidx[half], isem[half]).wait()
            fire_gathers(half)

    for half in range(2):
        pltpu.sync_copy(x_hbm.at[0, pl.ds(wid * NBT + half * 2, 2)],
                        xidx[half])
        fire_gathers(half)
    body(0, 0, outwait=False)
    body(0, 1, outwait=False)

    def lstep(l, carry):
        body(l, 0)
        body(l, 1)
        return carry

    lax.fori_loop(1, MAXLEN - 1, lstep, 0)
    body(MAXLEN - 1, 0, prefetch=False)
    body(MAXLEN - 1, 1, prefetch=False)
    for half in range(2):
        for dt in range(8):
            pltpu.make_async_copy(
                obuf[half].at[dt, pl.ds(0, 2), :, pl.ds(0, 128)],
                out_hbm.at[MAXLEN - 1, dt,
                           pl.ds(wid * NBT + half * 2, 2)],
                osem[half]).wait()


def kernel(X, c, token_table, category_table, pos_table):
    xT = X.T.reshape(MAXLEN, BATCH // 128, 128)
    c2 = c.reshape(BATCH // 128, 128)
    mesh = plsc.VectorSubcoreMesh(core_axis_name="c", subcore_axis_name="s")
    run = pl.kernel(
        _sc_body,
        mesh=mesh,
        compiler_params=pltpu.CompilerParams(use_tc_tiling_on_sc=False,
                                            needs_layout_passes=False),
        out_type=jax.ShapeDtypeStruct((MAXLEN, 8, 128, 8, 128), jnp.float32),
        scratch_types=[
            [pltpu.VMEM((2, 128), jnp.int32) for _ in range(2)],
            pltpu.VMEM((NBT, 128), jnp.int32),
            [pltpu.VMEM((HALF, EMBED_DIM), jnp.float32) for _ in range(2)],
            [pltpu.VMEM((8, 3, 8, 129), jnp.float32) for _ in range(2)],
            pltpu.VMEM((RPW, EMBED_DIM), jnp.float32),
            pltpu.VMEM((MAXLEN, EMBED_DIM), jnp.float32),
            [pltpu.SemaphoreType.DMA for _ in range(2)],
            [pltpu.SemaphoreType.DMA for _ in range(2)],
            [pltpu.SemaphoreType.DMA for _ in range(2)],
        ],
    )
    out5 = run(xT, c2, token_table, category_table, pos_table)
    return out5.transpose(2, 4, 0, 1, 3).reshape(BATCH, MAXLEN, EMBED_DIM)
